# baseline (speedup 1.0000x reference)
import jax
import jax.numpy as jnp
from jax.experimental import pallas as pl

_B = 64
_V = 100000
_K = 10000
_R = 8


def _gumbel_block(row0):
    rr = jax.lax.broadcasted_iota(jnp.int32, (_R, _V), 0) + row0
    p = rr * _V + jax.lax.broadcasted_iota(jnp.int32, (_R, _V), 1)

    def rotl(v, r):
        return (jax.lax.shift_left(v, jnp.int32(r))
                | jax.lax.shift_right_logical(v, jnp.int32(32 - r)))

    def four_rounds(x0, x1, rots):
        for r in rots:
            x0 = x0 + x1
            x1 = x0 ^ rotl(x1, r)
        return x0, x1

    ra = (13, 15, 26, 6)
    rb = (17, 29, 16, 24)
    ks0 = jnp.int32(0)
    ks1 = jnp.int32(42)
    ks2 = jnp.int32(42 ^ 0x1BD11BDA)
    x0 = jnp.zeros((_R, _V), jnp.int32) + ks0
    x1 = p + ks1
    x0, x1 = four_rounds(x0, x1, ra)
    x0, x1 = x0 + ks1, x1 + ks2 + jnp.int32(1)
    x0, x1 = four_rounds(x0, x1, rb)
    x0, x1 = x0 + ks2, x1 + ks0 + jnp.int32(2)
    x0, x1 = four_rounds(x0, x1, ra)
    x0, x1 = x0 + ks0, x1 + ks1 + jnp.int32(3)
    x0, x1 = four_rounds(x0, x1, rb)
    x0, x1 = x0 + ks1, x1 + ks2 + jnp.int32(4)
    x0, x1 = four_rounds(x0, x1, ra)
    x0, x1 = x0 + ks2, x1 + ks0 + jnp.int32(5)
    bits = x0 ^ x1
    u = jax.lax.bitcast_convert_type(
        jax.lax.shift_right_logical(bits, jnp.int32(9)) | jnp.int32(0x3F800000),
        jnp.float32) - 1.0
    u = jnp.maximum(u, jnp.float32(1.1754944e-38))
    return -jnp.log(-jnp.log(u))


def _body(x_ref, probs_ref, samp_ref):
    min32 = jnp.int32(-2147483648)
    one = jnp.int32(1)
    x = x_ref[...]
    b = jax.lax.bitcast_convert_type(x, jnp.int32)
    s = jnp.where(b < 0, ~b ^ min32, b)

    def f_to_key(f):
        bb = jax.lax.bitcast_convert_type(f, jnp.int32)
        return jnp.where(bb < 0, ~bb ^ min32, bb)

    def key_to_f(t):
        return jax.lax.bitcast_convert_type(
            jnp.where(t < 0, ~(t ^ min32), t), jnp.float32)

    xmax = jnp.max(x, axis=1, keepdims=True)
    xmin = jnp.min(x, axis=1, keepdims=True)
    mu = jnp.sum(x, axis=1, keepdims=True) * (1.0 / _V)
    var = jnp.sum(x * x, axis=1, keepdims=True) * (1.0 / _V) - mu * mu
    sd = jnp.sqrt(jnp.maximum(var, 1e-30))

    t_lo0 = f_to_key(xmin)
    c_lo0 = jnp.full((_R, 1), _V, jnp.int32)
    t_hi0 = f_to_key(xmax) + one
    c_hi0 = jnp.zeros((_R, 1), jnp.int32)
    nxt0 = mu + jnp.float32(1.2815516) * sd

    def live(c_lo, t_lo, t_hi):
        return (c_lo != _K) & ((t_hi - t_lo) != one)

    def cond(state):
        i, t_lo, c_lo, t_hi, c_hi, nxt = state
        return jnp.logical_and(
            i < 64, jnp.sum(live(c_lo, t_lo, t_hi).astype(jnp.int32)) > 0)

    def probe(state):
        i, t_lo, c_lo, t_hi, c_hi, nxt = state
        t_bis = t_lo + jax.lax.shift_right_logical(t_hi - t_lo, one)
        t_p = jnp.where(i < 24, f_to_key(nxt), t_bis)
        t_p = jnp.minimum(jnp.maximum(t_p, t_lo + one), t_hi - one)
        cnt = jnp.sum((s >= t_p).astype(jnp.int32), axis=1, keepdims=True)
        go_lo = cnt >= _K
        t_lo2 = jnp.where(go_lo, t_p, t_lo)
        c_lo2 = jnp.where(go_lo, cnt, c_lo)
        t_hi2 = jnp.where(go_lo, t_hi, t_p)
        c_hi2 = jnp.where(go_lo, c_hi, cnt)
        f_lo = key_to_f(t_lo2)
        f_hi = key_to_f(t_hi2)
        frac = ((c_lo2 - _K).astype(jnp.float32)
                / jnp.maximum((c_lo2 - c_hi2).astype(jnp.float32), 1.0))
        return (i + one, t_lo2, c_lo2, t_hi2, c_hi2,
                f_lo + (f_hi - f_lo) * frac)

    _, t_lo, c_lo, t_hi, c_hi, _ = jax.lax.while_loop(
        cond, probe, (jnp.int32(0), t_lo0, c_lo0, t_hi0, c_hi0, nxt0))
    mask = s >= t_lo

    e = jnp.where(mask, jnp.exp(x - xmax), 0.0)
    denom = jnp.sum(e, axis=1, keepdims=True)
    probs = e / denom
    probs_ref[...] = probs

    g = _gumbel_block(pl.program_id(0) * _R)
    v = jnp.log(probs + 1e-20) + g
    vm = jnp.max(v, axis=1, keepdims=True)
    iota = jax.lax.broadcasted_iota(jnp.int32, v.shape, 1)
    idx = jnp.min(jnp.where(v == vm, iota, jnp.int32(2**31 - 1)), axis=1,
                  keepdims=True)
    samp_ref[...] = idx


def kernel(logits):
    probs, samples = pl.pallas_call(
        _body,
        grid=(_B // _R,),
        in_specs=[
            pl.BlockSpec((_R, _V), lambda i: (i, 0)),
        ],
        out_specs=[
            pl.BlockSpec((_R, _V), lambda i: (i, 0)),
            pl.BlockSpec((_R, 1), lambda i: (i, 0)),
        ],
        out_shape=[
            jax.ShapeDtypeStruct((_B, _V), jnp.float32),
            jax.ShapeDtypeStruct((_B, 1), jnp.int32),
        ],
    )(logits)
    return samples, probs


# speedup vs baseline: 1.1521x; 1.1521x over previous
import jax
import jax.numpy as jnp
from jax.experimental import pallas as pl
from jax.experimental.pallas import tpu as pltpu

_B = 64
_V = 100000
_K = 10000
_R = 8


_CHUNK = 1024
_NFULL = _V // _CHUNK


def _gumbel_chunk(row0, col0, width):
    rr = jax.lax.broadcasted_iota(jnp.int32, (_R, width), 0) + row0
    p = rr * _V + jax.lax.broadcasted_iota(jnp.int32, (_R, width), 1) + col0

    def rotl(v, r):
        return (jax.lax.shift_left(v, jnp.int32(r))
                | jax.lax.shift_right_logical(v, jnp.int32(32 - r)))

    def four_rounds(x0, x1, rots):
        for r in rots:
            x0 = x0 + x1
            x1 = x0 ^ rotl(x1, r)
        return x0, x1

    ra = (13, 15, 26, 6)
    rb = (17, 29, 16, 24)
    ks0 = jnp.int32(0)
    ks1 = jnp.int32(42)
    ks2 = jnp.int32(42 ^ 0x1BD11BDA)
    x0 = jnp.zeros((_R, width), jnp.int32) + ks0
    x1 = p + ks1
    x0, x1 = four_rounds(x0, x1, ra)
    x0, x1 = x0 + ks1, x1 + ks2 + jnp.int32(1)
    x0, x1 = four_rounds(x0, x1, rb)
    x0, x1 = x0 + ks2, x1 + ks0 + jnp.int32(2)
    x0, x1 = four_rounds(x0, x1, ra)
    x0, x1 = x0 + ks0, x1 + ks1 + jnp.int32(3)
    x0, x1 = four_rounds(x0, x1, rb)
    x0, x1 = x0 + ks1, x1 + ks2 + jnp.int32(4)
    x0, x1 = four_rounds(x0, x1, ra)
    x0, x1 = x0 + ks2, x1 + ks0 + jnp.int32(5)
    bits = x0 ^ x1
    u = jax.lax.bitcast_convert_type(
        jax.lax.shift_right_logical(bits, jnp.int32(9)) | jnp.int32(0x3F800000),
        jnp.float32) - 1.0
    u = jnp.maximum(u, jnp.float32(1.1754944e-38))
    return -jnp.log(-jnp.log(u))


def _body(x_ref, probs_ref, samp_ref, g_ref):
    min32 = jnp.int32(-2147483648)
    one = jnp.int32(1)
    x = x_ref[...]
    b = jax.lax.bitcast_convert_type(x, jnp.int32)
    s = jnp.where(b < 0, ~b ^ min32, b)

    def f_to_key(f):
        bb = jax.lax.bitcast_convert_type(f, jnp.int32)
        return jnp.where(bb < 0, ~bb ^ min32, bb)

    def key_to_f(t):
        return jax.lax.bitcast_convert_type(
            jnp.where(t < 0, ~(t ^ min32), t), jnp.float32)

    xmax = jnp.max(x, axis=1, keepdims=True)
    xmin = jnp.min(x, axis=1, keepdims=True)
    mu = jnp.sum(x, axis=1, keepdims=True) * (1.0 / _V)
    var = jnp.sum(x * x, axis=1, keepdims=True) * (1.0 / _V) - mu * mu
    sd = jnp.sqrt(jnp.maximum(var, 1e-30))

    t_lo0 = f_to_key(xmin)
    c_lo0 = jnp.full((_R, 1), _V, jnp.int32)
    t_hi0 = f_to_key(xmax) + one
    c_hi0 = jnp.zeros((_R, 1), jnp.int32)
    nxt0 = mu + jnp.float32(1.2815516) * sd

    def live(c_lo, t_lo, t_hi):
        return (c_lo != _K) & ((t_hi - t_lo) != one)

    def cond(state):
        i, t_lo, c_lo, t_hi, c_hi, nxt = state
        return jnp.logical_and(
            i < 64, jnp.sum(live(c_lo, t_lo, t_hi).astype(jnp.int32)) > 0)

    def probe(state):
        i, t_lo, c_lo, t_hi, c_hi, nxt = state
        t_bis = t_lo + jax.lax.shift_right_logical(t_hi - t_lo, one)
        t_p = jnp.where(i < 24, f_to_key(nxt), t_bis)
        t_p = jnp.minimum(jnp.maximum(t_p, t_lo + one), t_hi - one)
        cnt = jnp.sum((s >= t_p).astype(jnp.int32), axis=1, keepdims=True)
        go_lo = cnt >= _K
        t_lo2 = jnp.where(go_lo, t_p, t_lo)
        c_lo2 = jnp.where(go_lo, cnt, c_lo)
        t_hi2 = jnp.where(go_lo, t_hi, t_p)
        c_hi2 = jnp.where(go_lo, c_hi, cnt)
        f_lo = key_to_f(t_lo2)
        f_hi = key_to_f(t_hi2)
        frac = ((c_lo2 - _K).astype(jnp.float32)
                / jnp.maximum((c_lo2 - c_hi2).astype(jnp.float32), 1.0))
        return (i + one, t_lo2, c_lo2, t_hi2, c_hi2,
                f_lo + (f_hi - f_lo) * frac)

    _, t_lo, c_lo, t_hi, c_hi, _ = jax.lax.while_loop(
        cond, probe, (jnp.int32(0), t_lo0, c_lo0, t_hi0, c_hi0, nxt0))
    mask = s >= t_lo

    e = jnp.where(mask, jnp.exp(x - xmax), 0.0)
    denom = jnp.sum(e, axis=1, keepdims=True)
    probs = e / denom
    probs_ref[...] = probs

    row0 = pl.program_id(0) * _R

    def gen(c, carry):
        g_ref[:, pl.ds(c * _CHUNK, _CHUNK)] = _gumbel_chunk(row0, c * _CHUNK,
                                                            _CHUNK)
        return carry

    jax.lax.fori_loop(0, _NFULL, gen, jnp.int32(0))
    g_ref[:, _NFULL * _CHUNK:_V] = _gumbel_chunk(row0, _NFULL * _CHUNK,
                                                 _V - _NFULL * _CHUNK)
    v = jnp.log(probs + 1e-20) + g_ref[...]
    vm = jnp.max(v, axis=1, keepdims=True)
    iota = jax.lax.broadcasted_iota(jnp.int32, v.shape, 1)
    idx = jnp.min(jnp.where(v == vm, iota, jnp.int32(2**31 - 1)), axis=1,
                  keepdims=True)
    samp_ref[...] = idx


def kernel(logits):
    probs, samples = pl.pallas_call(
        _body,
        grid=(_B // _R,),
        in_specs=[
            pl.BlockSpec((_R, _V), lambda i: (i, 0)),
        ],
        out_specs=[
            pl.BlockSpec((_R, _V), lambda i: (i, 0)),
            pl.BlockSpec((_R, 1), lambda i: (i, 0)),
        ],
        out_shape=[
            jax.ShapeDtypeStruct((_B, _V), jnp.float32),
            jax.ShapeDtypeStruct((_B, 1), jnp.int32),
        ],
        scratch_shapes=[pltpu.VMEM((_R, _V), jnp.float32)],
    )(logits)
    return samples, probs


# speedup vs baseline: 1.3934x; 1.2094x over previous
import jax
import jax.numpy as jnp
from jax.experimental import pallas as pl
from jax.experimental.pallas import tpu as pltpu

_B = 64
_V = 100000
_K = 10000
_R = 8


_CHUNK = 1024
_NFULL = _V // _CHUNK


def _gumbel_chunk(row0, col0, width):
    rr = jax.lax.broadcasted_iota(jnp.int32, (_R, width), 0) + row0
    p = rr * _V + jax.lax.broadcasted_iota(jnp.int32, (_R, width), 1) + col0

    def rotl(v, r):
        return (jax.lax.shift_left(v, jnp.int32(r))
                | jax.lax.shift_right_logical(v, jnp.int32(32 - r)))

    def four_rounds(x0, x1, rots):
        for r in rots:
            x0 = x0 + x1
            x1 = x0 ^ rotl(x1, r)
        return x0, x1

    ra = (13, 15, 26, 6)
    rb = (17, 29, 16, 24)
    ks0 = jnp.int32(0)
    ks1 = jnp.int32(42)
    ks2 = jnp.int32(42 ^ 0x1BD11BDA)
    x0 = jnp.zeros((_R, width), jnp.int32) + ks0
    x1 = p + ks1
    x0, x1 = four_rounds(x0, x1, ra)
    x0, x1 = x0 + ks1, x1 + ks2 + jnp.int32(1)
    x0, x1 = four_rounds(x0, x1, rb)
    x0, x1 = x0 + ks2, x1 + ks0 + jnp.int32(2)
    x0, x1 = four_rounds(x0, x1, ra)
    x0, x1 = x0 + ks0, x1 + ks1 + jnp.int32(3)
    x0, x1 = four_rounds(x0, x1, rb)
    x0, x1 = x0 + ks1, x1 + ks2 + jnp.int32(4)
    x0, x1 = four_rounds(x0, x1, ra)
    x0, x1 = x0 + ks2, x1 + ks0 + jnp.int32(5)
    bits = x0 ^ x1
    u = jax.lax.bitcast_convert_type(
        jax.lax.shift_right_logical(bits, jnp.int32(9)) | jnp.int32(0x3F800000),
        jnp.float32) - 1.0
    u = jnp.maximum(u, jnp.float32(1.1754944e-38))
    return -jnp.log(-jnp.log(u))


def _body(x_ref, probs_ref, samp_ref):
    min32 = jnp.int32(-2147483648)
    one = jnp.int32(1)
    x = x_ref[...]
    b = jax.lax.bitcast_convert_type(x, jnp.int32)
    s = jnp.where(b < 0, ~b ^ min32, b)

    def f_to_key(f):
        bb = jax.lax.bitcast_convert_type(f, jnp.int32)
        return jnp.where(bb < 0, ~bb ^ min32, bb)

    def key_to_f(t):
        return jax.lax.bitcast_convert_type(
            jnp.where(t < 0, ~(t ^ min32), t), jnp.float32)

    xmax = jnp.max(x, axis=1, keepdims=True)
    xmin = jnp.min(x, axis=1, keepdims=True)
    mu = jnp.sum(x, axis=1, keepdims=True) * (1.0 / _V)
    var = jnp.sum(x * x, axis=1, keepdims=True) * (1.0 / _V) - mu * mu
    sd = jnp.sqrt(jnp.maximum(var, 1e-30))

    t_lo0 = f_to_key(xmin)
    c_lo0 = jnp.full((_R, 1), _V, jnp.int32)
    t_hi0 = f_to_key(xmax) + one
    c_hi0 = jnp.zeros((_R, 1), jnp.int32)
    nxt0 = mu + jnp.float32(1.2815516) * sd

    def live(c_lo, t_lo, t_hi):
        return (c_lo != _K) & ((t_hi - t_lo) != one)

    def cond(state):
        i, t_lo, c_lo, t_hi, c_hi, nxt = state
        return jnp.logical_and(
            i < 64, jnp.sum(live(c_lo, t_lo, t_hi).astype(jnp.int32)) > 0)

    def probe(state):
        i, t_lo, c_lo, t_hi, c_hi, nxt = state
        t_bis = t_lo + jax.lax.shift_right_logical(t_hi - t_lo, one)
        t_p = jnp.where(i < 24, f_to_key(nxt), t_bis)
        t_p = jnp.minimum(jnp.maximum(t_p, t_lo + one), t_hi - one)
        cnt = jnp.sum((s >= t_p).astype(jnp.int32), axis=1, keepdims=True)
        go_lo = cnt >= _K
        t_lo2 = jnp.where(go_lo, t_p, t_lo)
        c_lo2 = jnp.where(go_lo, cnt, c_lo)
        t_hi2 = jnp.where(go_lo, t_hi, t_p)
        c_hi2 = jnp.where(go_lo, c_hi, cnt)
        f_lo = key_to_f(t_lo2)
        f_hi = key_to_f(t_hi2)
        frac = ((c_lo2 - _K).astype(jnp.float32)
                / jnp.maximum((c_lo2 - c_hi2).astype(jnp.float32), 1.0))
        return (i + one, t_lo2, c_lo2, t_hi2, c_hi2,
                f_lo + (f_hi - f_lo) * frac)

    _, t_lo, c_lo, t_hi, c_hi, _ = jax.lax.while_loop(
        cond, probe, (jnp.int32(0), t_lo0, c_lo0, t_hi0, c_hi0, nxt0))
    mask = s >= t_lo

    e = jnp.where(mask, jnp.exp(x - xmax), 0.0)
    denom = jnp.sum(e, axis=1, keepdims=True)
    probs = e / denom
    probs_ref[...] = probs

    row0 = pl.program_id(0) * _R

    def samp_chunk(c0, width):
        g = _gumbel_chunk(row0, c0, width)
        pc = probs[:, c0:c0 + width]
        v = jnp.log(pc + 1e-20) + g
        vm = jnp.max(v, axis=1, keepdims=True)
        io = jax.lax.broadcasted_iota(jnp.int32, v.shape, 1)
        ii = jnp.min(jnp.where(v == vm, io, jnp.int32(2**31 - 1)), axis=1,
                     keepdims=True) + c0
        return vm, ii

    run_m = jnp.full((_R, 1), -jnp.inf, jnp.float32)
    run_i = jnp.zeros((_R, 1), jnp.int32)
    for c in range(_NFULL):
        vm, ii = samp_chunk(c * _CHUNK, _CHUNK)
        upd = vm > run_m
        run_m = jnp.where(upd, vm, run_m)
        run_i = jnp.where(upd, ii, run_i)
    vm, ii = samp_chunk(_NFULL * _CHUNK, _V - _NFULL * _CHUNK)
    samp_ref[...] = jnp.where(vm > run_m, ii, run_i)


def kernel(logits):
    probs, samples = pl.pallas_call(
        _body,
        grid=(_B // _R,),
        in_specs=[
            pl.BlockSpec((_R, _V), lambda i: (i, 0)),
        ],
        out_specs=[
            pl.BlockSpec((_R, _V), lambda i: (i, 0)),
            pl.BlockSpec((_R, 1), lambda i: (i, 0)),
        ],
        out_shape=[
            jax.ShapeDtypeStruct((_B, _V), jnp.float32),
            jax.ShapeDtypeStruct((_B, 1), jnp.int32),
        ],
    )(logits)
    return samples, probs
